# baseline (speedup 1.0000x reference)
import functools

import jax
import jax.numpy as jnp
from jax import lax
from jax.experimental import pallas as pl
from jax.experimental.pallas import tpu as pltpu
from jax.experimental.pallas import tpu_sc as plsc

VOCAB = 49408
NUM_POS = 77
EMBED = 768
BATCH = 1024
ROWS = BATCH * NUM_POS

PIECES = 2
PIECE_BATCH = BATCH // PIECES
PIECE_ROWS = ROWS // PIECES

NUM_CORES = 2
NUM_SUBCORES = 16
NUM_WORKERS = NUM_CORES * NUM_SUBCORES
ROWS_PER_W = PIECE_ROWS // NUM_WORKERS
CHUNK = 16
NUM_CHUNKS = ROWS_PER_W // CHUNK
BUFS = 7

ITEMS_PER_BLK = 8
PIECE_GRID = PIECE_BATCH // ITEMS_PER_BLK
BLK_ROWS = ITEMS_PER_BLK * NUM_POS

_mesh = plsc.VectorSubcoreMesh(core_axis_name="c", subcore_axis_name="s")

_scratch = ([pltpu.VMEM((ROWS_PER_W,), jnp.int32)]
            + [pltpu.VMEM((CHUNK, EMBED), jnp.float32) for _ in range(BUFS)]
            + [pltpu.SemaphoreType.DMA for _ in range(2 * BUFS)])


@functools.partial(
    pl.kernel,
    mesh=_mesh,
    out_type=jax.ShapeDtypeStruct((PIECE_ROWS, EMBED), jnp.float32),
    scratch_types=_scratch,
)
def _gather_kernel(tok_hbm, table_hbm, out_hbm, idx_v, *rest):
    bufs = rest[:BUFS]
    gsems = rest[BUFS:2 * BUFS]
    ssems = rest[2 * BUFS:]
    wid = lax.axis_index("s") * NUM_CORES + lax.axis_index("c")
    wrow0 = wid * ROWS_PER_W

    def gather_desc(c, b):
        return pltpu.make_async_copy(
            table_hbm.at[idx_v.at[pl.ds(c * CHUNK, CHUNK)]], bufs[b], gsems[b])

    def store_desc(c, b):
        return pltpu.make_async_copy(
            bufs[b], out_hbm.at[pl.ds(wrow0 + c * CHUNK, CHUNK)], ssems[b])

    pltpu.sync_copy(tok_hbm.at[pl.ds(wrow0, ROWS_PER_W)], idx_v)

    for b in range(BUFS - 1):
        gather_desc(b, b).start()

    def group_body(g, _):
        for b in range(BUFS):
            c = g * BUFS + b
            gather_desc(c, b).wait()

            @pl.when(c >= 1)
            def _():
                store_desc(c - 1, (b - 1) % BUFS).wait()

            @pl.when(c + BUFS - 1 < NUM_CHUNKS)
            def _():
                gather_desc(c + BUFS - 1, (b - 1) % BUFS).start()

            store_desc(c, b).start()
        return 0

    lax.fori_loop(0, NUM_CHUNKS // BUFS, group_body, 0)
    store_desc(NUM_CHUNKS - 1, (NUM_CHUNKS - 1) % BUFS).wait()


def _add_body(g_ref, p_ref, o_ref):
    for i in range(ITEMS_PER_BLK):
        s = slice(i * NUM_POS, (i + 1) * NUM_POS)
        o_ref[i] = g_ref[s] + p_ref[s]


def _add_body_aliased(g_ref, p_ref, prev_ref, o_ref):
    del prev_ref
    _add_body(g_ref, p_ref, o_ref)


_out_sds = jax.ShapeDtypeStruct((BATCH, NUM_POS, EMBED), jnp.float32)

_add_first = pl.pallas_call(
    _add_body,
    grid=(PIECE_GRID,),
    in_specs=[
        pl.BlockSpec((BLK_ROWS, EMBED), lambda c: (c, 0)),
        pl.BlockSpec((BLK_ROWS, EMBED), lambda c: (0, 0)),
    ],
    out_specs=pl.BlockSpec((ITEMS_PER_BLK, NUM_POS, EMBED),
                           lambda c: (c, 0, 0)),
    out_shape=_out_sds,
)

_add_second = pl.pallas_call(
    _add_body_aliased,
    grid=(PIECE_GRID,),
    in_specs=[
        pl.BlockSpec((BLK_ROWS, EMBED), lambda c: (c, 0)),
        pl.BlockSpec((BLK_ROWS, EMBED), lambda c: (0, 0)),
        pl.BlockSpec(memory_space=pl.ANY),
    ],
    out_specs=pl.BlockSpec((ITEMS_PER_BLK, NUM_POS, EMBED),
                           lambda c: (c + PIECE_GRID, 0, 0)),
    out_shape=_out_sds,
    input_output_aliases={2: 0},
)


def kernel(input_tokens, token_table, position_table):
    tok = input_tokens.astype(jnp.int32).reshape(ROWS)
    pos_rep = jnp.tile(position_table, (ITEMS_PER_BLK, 1))
    ga = _gather_kernel(tok[:PIECE_ROWS], token_table)
    gb = _gather_kernel(tok[PIECE_ROWS:], token_table)
    out = _add_first(ga, pos_rep)
    out = _add_second(gb, pos_rep, out)
    return out


# speedup vs baseline: 1.0051x; 1.0051x over previous
import functools

import jax
import jax.numpy as jnp
from jax import lax
from jax.experimental import pallas as pl
from jax.experimental.pallas import tpu as pltpu
from jax.experimental.pallas import tpu_sc as plsc

VOCAB = 49408
NUM_POS = 77
EMBED = 768
BATCH = 1024
ROWS = BATCH * NUM_POS

PIECES = 4
PIECE_BATCH = BATCH // PIECES
PIECE_ROWS = ROWS // PIECES

NUM_CORES = 2
NUM_SUBCORES = 16
NUM_WORKERS = NUM_CORES * NUM_SUBCORES
ROWS_PER_W = PIECE_ROWS // NUM_WORKERS
CHUNK = 8
NUM_CHUNKS = ROWS_PER_W // CHUNK
BUFS = 7

ITEMS_PER_BLK = 8
PIECE_GRID = PIECE_BATCH // ITEMS_PER_BLK
BLK_ROWS = ITEMS_PER_BLK * NUM_POS

_mesh = plsc.VectorSubcoreMesh(core_axis_name="c", subcore_axis_name="s")

_scratch = ([pltpu.VMEM((ROWS_PER_W,), jnp.int32)]
            + [pltpu.VMEM((CHUNK, EMBED), jnp.float32) for _ in range(BUFS)]
            + [pltpu.SemaphoreType.DMA for _ in range(2 * BUFS)])


@functools.partial(
    pl.kernel,
    mesh=_mesh,
    out_type=jax.ShapeDtypeStruct((PIECE_ROWS, EMBED), jnp.float32),
    scratch_types=_scratch,
)
def _gather_kernel(tok_hbm, table_hbm, out_hbm, idx_v, *rest):
    bufs = rest[:BUFS]
    gsems = rest[BUFS:2 * BUFS]
    ssems = rest[2 * BUFS:]
    wid = lax.axis_index("s") * NUM_CORES + lax.axis_index("c")
    wrow0 = wid * ROWS_PER_W

    def gather_desc(c, b):
        return pltpu.make_async_copy(
            table_hbm.at[idx_v.at[pl.ds(c * CHUNK, CHUNK)]], bufs[b], gsems[b])

    def store_desc(c, b):
        return pltpu.make_async_copy(
            bufs[b], out_hbm.at[pl.ds(wrow0 + c * CHUNK, CHUNK)], ssems[b])

    pltpu.sync_copy(tok_hbm.at[pl.ds(wrow0, ROWS_PER_W)], idx_v)

    for b in range(BUFS - 1):
        gather_desc(b, b).start()

    def group_body(g, _):
        for b in range(BUFS):
            c = g * BUFS + b
            gather_desc(c, b).wait()

            @pl.when(c >= 1)
            def _():
                store_desc(c - 1, (b - 1) % BUFS).wait()

            @pl.when(c + BUFS - 1 < NUM_CHUNKS)
            def _():
                gather_desc(c + BUFS - 1, (b - 1) % BUFS).start()

            store_desc(c, b).start()
        return 0

    lax.fori_loop(0, NUM_CHUNKS // BUFS, group_body, 0)
    store_desc(NUM_CHUNKS - 1, (NUM_CHUNKS - 1) % BUFS).wait()


def _add_rows(g_ref, p_ref, o_ref):
    for i in range(ITEMS_PER_BLK):
        s = slice(i * NUM_POS, (i + 1) * NUM_POS)
        o_ref[i] = g_ref[s] + p_ref[s]


def _add_body_first(g_ref, p_ref, o_ref):
    _add_rows(g_ref, p_ref, o_ref)


def _add_body_next(g_ref, p_ref, prev_ref, o_ref):
    del prev_ref
    _add_rows(g_ref, p_ref, o_ref)


_out_sds = jax.ShapeDtypeStruct((BATCH, NUM_POS, EMBED), jnp.float32)


def _make_add(piece, first):
    body = _add_body_first if first else _add_body_next
    in_specs = [
        pl.BlockSpec((BLK_ROWS, EMBED), lambda c: (c, 0)),
        pl.BlockSpec((BLK_ROWS, EMBED), lambda c: (0, 0)),
    ]
    kwargs = {}
    if not first:
        in_specs.append(pl.BlockSpec(memory_space=pl.ANY))
        kwargs["input_output_aliases"] = {2: 0}
    return pl.pallas_call(
        body,
        grid=(PIECE_GRID,),
        in_specs=in_specs,
        out_specs=pl.BlockSpec(
            (ITEMS_PER_BLK, NUM_POS, EMBED),
            functools.partial(lambda pc, c: (pc * PIECE_GRID + c, 0, 0), piece)),
        out_shape=_out_sds,
        **kwargs,
    )


_adds = [_make_add(p, p == 0) for p in range(PIECES)]


def kernel(input_tokens, token_table, position_table):
    tok = input_tokens.astype(jnp.int32).reshape(ROWS)
    pos_rep = jnp.tile(position_table, (ITEMS_PER_BLK, 1))
    gathered = [
        _gather_kernel(tok[p * PIECE_ROWS:(p + 1) * PIECE_ROWS], token_table)
        for p in range(PIECES)
    ]
    out = _adds[0](gathered[0], pos_rep)
    for p in range(1, PIECES):
        out = _adds[p](gathered[p], pos_rep, out)
    return out
